# Initial kernel scaffold; baseline (speedup 1.0000x reference)
import jax
import jax.numpy as jnp
from jax.experimental import pallas as pl


def kernel(h, x, edges, edge_attr, We1, be1, We2, be2, Wn1, bn1, Wn2, bn2):
    raise NotImplementedError("write your pallas kernel here")


# speedup vs baseline: 4.1569x; 4.1569x over previous
import functools

import jax
import jax.numpy as jnp
from jax import lax
from jax.experimental import pallas as pl
from jax.experimental.pallas import tpu as pltpu
from jax.experimental.pallas import tpu_sc as plsc

N = 10000
E = 320000
D = 128

NC = 2
NS = 16
NW = NC * NS
EDGES_PER_W = E // NW
CHUNK = 400
NCHUNK = EDGES_PER_W // CHUNK
NPAD = 10240
ROWS_PER_TILE = NPAD // NS
WB_CHUNK = 160
SCHUNK = 200
NSCHUNK = EDGES_PER_W // SCHUNK

_sc_mesh = plsc.VectorSubcoreMesh(core_axis_name="c", subcore_axis_name="s")


def _wid(c, s):
    return s * NC + c


@functools.partial(
    pl.kernel,
    mesh=_sc_mesh,
    out_type=[
        jax.ShapeDtypeStruct((E, D), jnp.float32),
        jax.ShapeDtypeStruct((E, D), jnp.float32),
        jax.ShapeDtypeStruct((E,), jnp.float32),
    ],
    scratch_types=[
        pltpu.VMEM((CHUNK,), jnp.int32),
        pltpu.VMEM((CHUNK,), jnp.int32),
        pltpu.VMEM((CHUNK, D), jnp.float32),
        pltpu.VMEM((CHUNK,), jnp.float32),
        pltpu.VMEM((4 * N,), jnp.float32),
        pltpu.SemaphoreType.DMA,
    ],
    compiler_params=pltpu.CompilerParams(needs_layout_passes=False),
)
def _sc_gather(h_hbm, x4_hbm, row_hbm, col_hbm,
               hr_hbm, hc_hbm, rad_hbm,
               idx_r, idx_c, hbuf, radbuf, xloc, sem):
    c = lax.axis_index("c")
    s = lax.axis_index("s")
    base = _wid(c, s) * EDGES_PER_W

    pltpu.sync_copy(x4_hbm, xloc)

    def body(j, _):
        off = base + j * CHUNK
        pltpu.sync_copy(row_hbm.at[pl.ds(off, CHUNK)], idx_r)
        pltpu.sync_copy(col_hbm.at[pl.ds(off, CHUNK)], idx_c)
        pltpu.async_copy(h_hbm.at[idx_r], hbuf, sem).wait()
        pltpu.sync_copy(hbuf, hr_hbm.at[pl.ds(off, CHUNK)])
        pltpu.async_copy(h_hbm.at[idx_c], hbuf, sem).wait()
        pltpu.sync_copy(hbuf, hc_hbm.at[pl.ds(off, CHUNK)])

        def rgroup(g, _2):
            ir = idx_r[pl.ds(g * 16, 16)] * 4
            ic = idx_c[pl.ds(g * 16, 16)] * 4
            acc = jnp.zeros((16,), jnp.float32)
            for k in range(3):
                xrk = plsc.load_gather(xloc, [ir + k])
                xck = plsc.load_gather(xloc, [ic + k])
                dk = xrk - xck
                acc = acc + dk * dk
            radbuf[pl.ds(g * 16, 16)] = acc
            return _2

        lax.fori_loop(0, CHUNK // 16, rgroup, 0)
        pltpu.sync_copy(radbuf, rad_hbm.at[pl.ds(off, CHUNK)])
        return _

    lax.fori_loop(0, NCHUNK, body, 0)


def _silu(v):
    return v * jax.nn.sigmoid(v)


def _edge_mlp_body(hr, hc, rad, ea, wa, wb, wrad, wea, be1, we2, be2, m_out):
    radial = rad[...]
    acc = jax.lax.dot_general(hr[...], wa[...], (((1,), (0,)), ((), ())),
                              preferred_element_type=jnp.float32)
    acc += jax.lax.dot_general(hc[...], wb[...], (((1,), (0,)), ((), ())),
                               preferred_element_type=jnp.float32)
    acc += jax.lax.dot_general(ea[...], wea[...], (((1,), (0,)), ((), ())),
                               preferred_element_type=jnp.float32)
    acc += radial * wrad[...]
    acc += be1[...]
    m1 = _silu(acc)
    m2 = jax.lax.dot_general(m1, we2[...], (((1,), (0,)), ((), ())),
                             preferred_element_type=jnp.float32)
    m2 += be2[...]
    m_out[...] = _silu(m2)


def _edge_mlp(hr, hc, rad, ea, wa, wb, wrad, wea, be1, we2, be2, blk=2000):
    grid = (E // blk,)
    return pl.pallas_call(
        _edge_mlp_body,
        grid=grid,
        in_specs=[
            pl.BlockSpec((blk, D), lambda i: (i, 0)),
            pl.BlockSpec((blk, D), lambda i: (i, 0)),
            pl.BlockSpec((blk, 1), lambda i: (i, 0)),
            pl.BlockSpec((blk, 16), lambda i: (i, 0)),
            pl.BlockSpec((D, D), lambda i: (0, 0)),
            pl.BlockSpec((D, D), lambda i: (0, 0)),
            pl.BlockSpec((1, D), lambda i: (0, 0)),
            pl.BlockSpec((16, D), lambda i: (0, 0)),
            pl.BlockSpec((1, D), lambda i: (0, 0)),
            pl.BlockSpec((D, D), lambda i: (0, 0)),
            pl.BlockSpec((1, D), lambda i: (0, 0)),
        ],
        out_specs=pl.BlockSpec((blk, D), lambda i: (i, 0)),
        out_shape=jax.ShapeDtypeStruct((E, D), jnp.float32),
    )(hr, hc, rad, ea, wa, wb, wrad, wea, be1, we2, be2)


@functools.partial(
    pl.kernel,
    mesh=_sc_mesh,
    out_type=jax.ShapeDtypeStruct((NC, NPAD, D), jnp.float32),
    scratch_types=[
        pltpu.VMEM((SCHUNK,), jnp.int32),
        pltpu.VMEM((SCHUNK, D), jnp.float32),
        pltpu.VMEM_SHARED((NPAD, D), jnp.float32),
    ],
)
def _sc_scatter(m_hbm, row_hbm, zeros_hbm, agg_hbm, idx_v, mbuf, agg_sh):
    c = lax.axis_index("c")
    s = lax.axis_index("s")
    base = _wid(c, s) * EDGES_PER_W

    pltpu.sync_copy(zeros_hbm, agg_sh.at[pl.ds(s * ROWS_PER_TILE, ROWS_PER_TILE)])
    plsc.subcore_barrier()

    def body(j, _):
        off = base + j * SCHUNK
        pltpu.sync_copy(row_hbm.at[pl.ds(off, SCHUNK)], idx_v)
        pltpu.sync_copy(m_hbm.at[pl.ds(off, SCHUNK)], mbuf)
        pltpu.sync_copy(mbuf, agg_sh.at[idx_v], add=True)
        return _

    lax.fori_loop(0, NSCHUNK, body, 0)

    plsc.subcore_barrier()
    def wb(j, _):
        r = s * ROWS_PER_TILE + j * WB_CHUNK
        pltpu.sync_copy(agg_sh.at[pl.ds(r, WB_CHUNK)], mbuf.at[pl.ds(0, WB_CHUNK)])
        pltpu.sync_copy(mbuf.at[pl.ds(0, WB_CHUNK)], agg_hbm.at[c, pl.ds(r, WB_CHUNK)])
        return _

    lax.fori_loop(0, ROWS_PER_TILE // WB_CHUNK, wb, 0)


def _node_mlp_body(h, agg, wna, wnb, bn1, wn2, bn2, out):
    a = agg[0] + agg[1]
    acc = jax.lax.dot_general(h[...], wna[...], (((1,), (0,)), ((), ())),
                              preferred_element_type=jnp.float32)
    acc += jax.lax.dot_general(a, wnb[...], (((1,), (0,)), ((), ())),
                               preferred_element_type=jnp.float32)
    acc += bn1[...]
    o = _silu(acc)
    o = jax.lax.dot_general(o, wn2[...], (((1,), (0,)), ((), ())),
                            preferred_element_type=jnp.float32)
    out[...] = o + bn2[...] + h[...]


def _node_mlp(h, agg2, wna, wnb, bn1, wn2, bn2, blk=2000):
    grid = (N // blk,)
    return pl.pallas_call(
        _node_mlp_body,
        grid=grid,
        in_specs=[
            pl.BlockSpec((blk, D), lambda i: (i, 0)),
            pl.BlockSpec((NC, blk, D), lambda i: (0, i, 0)),
            pl.BlockSpec((D, D), lambda i: (0, 0)),
            pl.BlockSpec((D, D), lambda i: (0, 0)),
            pl.BlockSpec((1, D), lambda i: (0, 0)),
            pl.BlockSpec((D, D), lambda i: (0, 0)),
            pl.BlockSpec((1, D), lambda i: (0, 0)),
        ],
        out_specs=pl.BlockSpec((blk, D), lambda i: (i, 0)),
        out_shape=jax.ShapeDtypeStruct((N, D), jnp.float32),
    )(h, agg2, wna, wnb, bn1, wn2, bn2)


def kernel(h, x, edges, edge_attr, We1, be1, We2, be2, Wn1, bn1, Wn2, bn2):
    row = edges[0].astype(jnp.int32)
    col = edges[1].astype(jnp.int32)
    x4flat = jnp.pad(x, ((0, 0), (0, 1))).reshape(-1)

    hr, hc, rad = _sc_gather(h, x4flat, row, col)

    wa = We1[0:D]
    wb = We1[D:2 * D]
    wrad = We1[2 * D:2 * D + 1]
    wea = We1[2 * D + 1:]
    m = _edge_mlp(hr, hc, rad.reshape(E, 1), edge_attr, wa, wb, wrad, wea,
                  be1.reshape(1, D), We2, be2.reshape(1, D))

    zeros = jnp.zeros((ROWS_PER_TILE, D), jnp.float32)
    agg2 = _sc_scatter(m, row, zeros)

    h_out = _node_mlp(h, agg2, Wn1[0:D], Wn1[D:], bn1.reshape(1, D),
                      Wn2, bn2.reshape(1, D))
    return (h_out, x, edge_attr)
